# baseline (speedup 1.0000x reference)
import functools

import numpy as np
import jax
import jax.numpy as jnp
from jax import lax
from jax.experimental import pallas as pl
from jax.experimental.pallas import tpu as pltpu
from jax.experimental.pallas import tpu_sc as plsc

N, M, F, A = 16384, 8, 32, 16
MF = M * F
MA = M * A
D = MF // 2
NE = N * M

_f32 = jnp.float32

_S = np.zeros((MF, M), np.float32)
_B8 = np.zeros((M, MF), np.float32)
_KS = np.zeros((MF, F), np.float32)
for _k in range(M):
    _S[_k * F:(_k + 1) * F, _k] = 1.0
    _B8[_k, _k * F:(_k + 1) * F] = 1.0
    _KS[_k * F:(_k + 1) * F, :] = np.eye(F, dtype=np.float32)


def _phase0_body(e_ref, w_ref, o_ref):
    e2 = e_ref[...].reshape(e_ref.shape[0], MF)
    y = jnp.dot(e2, w_ref[...], preferred_element_type=_f32)
    lo = lax.bitcast_convert_type(y[:, :MF // 2].astype(jnp.bfloat16),
                                  jnp.uint16).astype(jnp.uint32)
    hi = lax.bitcast_convert_type(y[:, MF // 2:].astype(jnp.bfloat16),
                                  jnp.uint16).astype(jnp.uint32)
    o_ref[...] = lax.bitcast_convert_type(lo | (hi << 16), jnp.int32)


def _apack_body(a_ref, o_ref):
    aa = a_ref[...].reshape(a_ref.shape[0], MA)
    lo = lax.bitcast_convert_type(aa[:, :MA // 2].astype(jnp.bfloat16),
                                  jnp.uint16).astype(jnp.uint32)
    hi = lax.bitcast_convert_type(aa[:, MA // 2:].astype(jnp.bfloat16),
                                  jnp.uint16).astype(jnp.uint32)
    o_ref[...] = lax.bitcast_convert_type(lo | (hi << 16), jnp.int32)


def _main_body(e_ref, gz_ref, ang_ref, w1t_ref, bd3_ref, s_ref, b8g_ref,
               ks_ref, scm_ref, cv_ref, o_ref):
    e = e_ref[...]
    gzu = lax.bitcast_convert_type(gz_ref[...], jnp.uint32)
    gz = jnp.concatenate(
        [lax.bitcast_convert_type(gzu << 16, _f32),
         lax.bitcast_convert_type(gzu & jnp.uint32(0xFFFF0000), _f32)],
        axis=1)
    cv = cv_ref[...]
    blin_t = cv[0:1, :]
    b1_t = cv[2:3, :]
    g2 = cv[3:4, 0:F]
    b2 = cv[3:4, F:2 * F]
    b_att = cv[3:4, 2 * F:2 * F + 1]

    angu = lax.bitcast_convert_type(ang_ref[...], jnp.uint32)
    ang = jnp.concatenate(
        [lax.bitcast_convert_type(angu << 16, _f32),
         lax.bitcast_convert_type(angu & jnp.uint32(0xFFFF0000), _f32)],
        axis=1)
    t = (jnp.dot(e, w1t_ref[...], preferred_element_type=_f32)
         + jnp.dot(ang, bd3_ref[...], preferred_element_type=_f32))
    lin = gz + t[:, :MF] + blin_t
    att = (t[:, MF:MF + M] + b_att
           + jnp.dot(gz, scm_ref[...], preferred_element_type=_f32))
    att = jnp.where(att >= 0, att, 0.01 * att)
    ex = jnp.exp(att)
    alpha = ex / jnp.sum(ex, axis=1, keepdims=True)

    b8g = b8g_ref[...]
    x = jnp.dot(alpha, b8g[M:M + M, :], preferred_element_type=_f32) * lin
    s = s_ref[...]
    mu = jnp.dot(x, s, preferred_element_type=_f32) * (1.0 / F)
    msq = jnp.dot(x * x, s, preferred_element_type=_f32) * (1.0 / F)
    inv = lax.rsqrt(msq - mu * mu + 1e-5)
    a_sc = jnp.dot(inv, b8g[:M, :], preferred_element_type=_f32)
    c_sc = b1_t - jnp.dot(mu * inv, b8g[:M, :], preferred_element_type=_f32)
    xn = x * a_sc + c_sc
    x = jnp.log1p(jnp.exp(xn))

    y = e + jnp.dot(x, ks_ref[...], preferred_element_type=_f32)
    mu2 = jnp.mean(y, axis=1, keepdims=True)
    yc = y - mu2
    var2 = jnp.mean(yc * yc, axis=1, keepdims=True)
    y = yc * lax.rsqrt(var2 + 1e-5) * g2 + b2
    o_ref[...] = jnp.log1p(jnp.exp(y))


def _sc_gather(z, idx):
    nidx = idx.shape[0]
    zdt = z.dtype
    info = plsc.get_sparse_core_info()
    nw = info.num_cores * info.num_subcores
    b_per_w = nidx // nw
    ch = 128
    n_pair = b_per_w // (2 * ch)
    mesh = plsc.VectorSubcoreMesh(core_axis_name="c", subcore_axis_name="s")

    @functools.partial(
        pl.kernel, mesh=mesh,
        out_type=jax.ShapeDtypeStruct((nidx, D), zdt),
        scratch_types=[
            pltpu.VMEM((b_per_w,), jnp.int32),
            pltpu.VMEM((ch, D), zdt),
            pltpu.VMEM((ch, D), zdt),
            pltpu.SemaphoreType.DMA,
            pltpu.SemaphoreType.DMA,
        ],
    )
    def gather_kernel(z_hbm, idx_hbm, out_hbm, idx_v, row0_v, row1_v, sem0, sem1):
        wid = lax.axis_index("s") * info.num_cores + lax.axis_index("c")
        base = pl.multiple_of(wid * b_per_w, b_per_w)
        pltpu.sync_copy(idx_hbm.at[pl.ds(base, b_per_w)], idx_v)

        def body(i, carry):
            o = pl.multiple_of(i * 2 * ch, 2 * ch)
            cp0 = pltpu.async_copy(
                z_hbm.at[idx_v.at[pl.ds(o, ch)]], row0_v, sem0)
            cp1 = pltpu.async_copy(
                z_hbm.at[idx_v.at[pl.ds(o + ch, ch)]], row1_v, sem1)
            cp0.wait()
            pltpu.sync_copy(row0_v, out_hbm.at[pl.ds(base + o, ch)])
            cp1.wait()
            pltpu.sync_copy(row1_v, out_hbm.at[pl.ds(base + o + ch, ch)])
            return carry

        lax.fori_loop(0, n_pair, body, 0)

    return gather_kernel(z, idx)


def kernel(edge_fea, angle_fea, nbr_idx, W_lin, b_lin, W_att, b_att, g1, b1, g2, b2):
    W1, W2, W3 = W_lin[:F], W_lin[F:2 * F], W_lin[2 * F:]
    Wa1, Wa2, Wa3 = W_att[:F], W_att[F:2 * F], W_att[2 * F:]

    eye8 = jnp.eye(M, dtype=_f32)

    bd2 = jnp.kron(eye8, W2)

    c2 = jnp.linalg.solve(W2, Wa2)
    c2 = c2 + jnp.linalg.solve(W2, Wa2 - W2 @ c2)
    scm = jnp.kron(eye8, c2)

    bd3 = jnp.concatenate([jnp.kron(eye8, W3), jnp.kron(eye8, Wa3)], axis=1)

    w1t = jnp.concatenate([jnp.tile(W1, (1, M)), jnp.tile(Wa1, (1, M))], axis=1)

    b8g = jnp.concatenate([_B8 * jnp.tile(g1, M)[None, :], jnp.asarray(_B8)], axis=0)

    row3 = jnp.concatenate([g2, b2, b_att, jnp.zeros((MF - 2 * F - 1,), _f32)])
    cv = jnp.stack([jnp.tile(b_lin, M), jnp.tile(g1, M), jnp.tile(b1, M), row3])

    e_flat = edge_fea.reshape(NE, F)
    ang3 = angle_fea.reshape(NE, M, A)
    idx_flat = nbr_idx.reshape(NE)

    bn = 2048
    z = pl.pallas_call(
        _phase0_body,
        grid=(N // bn,),
        in_specs=[
            pl.BlockSpec((bn, M, F), lambda i: (i, 0, 0)),
            pl.BlockSpec((MF, MF), lambda i: (0, 0)),
        ],
        out_specs=pl.BlockSpec((bn, D), lambda i: (i, 0)),
        out_shape=jax.ShapeDtypeStruct((N, D), jnp.int32),
    )(edge_fea, bd2)

    gz = _sc_gather(z, idx_flat)

    bp = 2048
    angp = pl.pallas_call(
        _apack_body,
        grid=(NE // bp,),
        in_specs=[pl.BlockSpec((bp, M, A), lambda i: (i, 0, 0))],
        out_specs=pl.BlockSpec((bp, MA // 2), lambda i: (i, 0)),
        out_shape=jax.ShapeDtypeStruct((NE, MA // 2), jnp.int32),
    )(ang3)

    r = 4096
    out = pl.pallas_call(
        _main_body,
        grid=(NE // r,),
        in_specs=[
            pl.BlockSpec((r, F), lambda i: (i, 0)),
            pl.BlockSpec((r, D), lambda i: (i, 0)),
            pl.BlockSpec((r, MA // 2), lambda i: (i, 0)),
            pl.BlockSpec((F, MF + M), lambda i: (0, 0)),
            pl.BlockSpec((MA, MF + M), lambda i: (0, 0)),
            pl.BlockSpec((MF, M), lambda i: (0, 0)),
            pl.BlockSpec((2 * M, MF), lambda i: (0, 0)),
            pl.BlockSpec((MF, F), lambda i: (0, 0)),
            pl.BlockSpec((MF, M), lambda i: (0, 0)),
            pl.BlockSpec((4, MF), lambda i: (0, 0)),
        ],
        out_specs=pl.BlockSpec((r, F), lambda i: (i, 0)),
        out_shape=jax.ShapeDtypeStruct((NE, F), _f32),
    )(e_flat, gz, angp, w1t, bd3, _S, b8g, _KS, scm, cv)

    return out.reshape(N, M, F)


# speedup vs baseline: 1.1739x; 1.1739x over previous
import functools

import numpy as np
import jax
import jax.numpy as jnp
from jax import lax
from jax.experimental import pallas as pl
from jax.experimental.pallas import tpu as pltpu
from jax.experimental.pallas import tpu_sc as plsc

N, M, F, A = 16384, 8, 32, 16
MF = M * F
MA = M * A
D = MF // 2
NE = N * M

_f32 = jnp.float32

_S = np.zeros((MF, M), np.float32)
_B8 = np.zeros((M, MF), np.float32)
_KS = np.zeros((MF, F), np.float32)
for _k in range(M):
    _S[_k * F:(_k + 1) * F, _k] = 1.0
    _B8[_k, _k * F:(_k + 1) * F] = 1.0
    _KS[_k * F:(_k + 1) * F, :] = np.eye(F, dtype=np.float32)


def _phase0_body(e_ref, w_ref, o_ref):
    e2 = e_ref[...].reshape(e_ref.shape[0], MF)
    y = jnp.dot(e2, w_ref[...], preferred_element_type=_f32)
    lo = lax.bitcast_convert_type(y[:, :MF // 2].astype(jnp.bfloat16),
                                  jnp.uint16).astype(jnp.uint32)
    hi = lax.bitcast_convert_type(y[:, MF // 2:].astype(jnp.bfloat16),
                                  jnp.uint16).astype(jnp.uint32)
    o_ref[...] = lax.bitcast_convert_type(lo | (hi << 16), jnp.int32)


def _main_body(e_ref, gz_ref, ang_ref, w1t_ref, bd3_ref, s_ref, b8g_ref,
               ks_ref, scm_ref, cv_ref, o_ref):
    e = e_ref[...]
    gzu = lax.bitcast_convert_type(gz_ref[...], jnp.uint32)
    gz = jnp.concatenate(
        [lax.bitcast_convert_type(gzu << 16, _f32),
         lax.bitcast_convert_type(gzu & jnp.uint32(0xFFFF0000), _f32)],
        axis=1)
    cv = cv_ref[...]
    blin_t = cv[0:1, :]
    b1_t = cv[2:3, :]
    g2 = cv[3:4, 0:F]
    b2 = cv[3:4, F:2 * F]
    b_att = cv[3:4, 2 * F:2 * F + 1]

    ang = ang_ref[...].reshape(e.shape[0], MA)
    t = (jnp.dot(e, w1t_ref[...], preferred_element_type=_f32)
         + jnp.dot(ang, bd3_ref[...], preferred_element_type=_f32))
    lin = gz + t[:, :MF] + blin_t
    att = (t[:, MF:MF + M] + b_att
           + jnp.dot(gz, scm_ref[...], preferred_element_type=_f32))
    att = jnp.where(att >= 0, att, 0.01 * att)
    ex = jnp.exp(att)
    alpha = ex / jnp.sum(ex, axis=1, keepdims=True)

    b8g = b8g_ref[...]
    x = jnp.dot(alpha, b8g[M:M + M, :], preferred_element_type=_f32) * lin
    s = s_ref[...]
    mu = jnp.dot(x, s, preferred_element_type=_f32) * (1.0 / F)
    msq = jnp.dot(x * x, s, preferred_element_type=_f32) * (1.0 / F)
    inv = lax.rsqrt(msq - mu * mu + 1e-5)
    a_sc = jnp.dot(inv, b8g[:M, :], preferred_element_type=_f32)
    c_sc = b1_t - jnp.dot(mu * inv, b8g[:M, :], preferred_element_type=_f32)
    xn = x * a_sc + c_sc
    x = jnp.log1p(jnp.exp(xn))

    y = e + jnp.dot(x, ks_ref[...], preferred_element_type=_f32)
    mu2 = jnp.mean(y, axis=1, keepdims=True)
    yc = y - mu2
    var2 = jnp.mean(yc * yc, axis=1, keepdims=True)
    y = yc * lax.rsqrt(var2 + 1e-5) * g2 + b2
    o_ref[...] = jnp.log1p(jnp.exp(y))


def _sc_gather(z, idx):
    nidx = idx.shape[0]
    zdt = z.dtype
    info = plsc.get_sparse_core_info()
    nw = info.num_cores * info.num_subcores
    b_per_w = nidx // nw
    ch = 128
    n_pair = b_per_w // (2 * ch)
    mesh = plsc.VectorSubcoreMesh(core_axis_name="c", subcore_axis_name="s")

    @functools.partial(
        pl.kernel, mesh=mesh,
        out_type=jax.ShapeDtypeStruct((nidx, D), zdt),
        scratch_types=[
            pltpu.VMEM((b_per_w,), jnp.int32),
            pltpu.VMEM((ch, D), zdt),
            pltpu.VMEM((ch, D), zdt),
            pltpu.SemaphoreType.DMA,
            pltpu.SemaphoreType.DMA,
        ],
    )
    def gather_kernel(z_hbm, idx_hbm, out_hbm, idx_v, row0_v, row1_v, sem0, sem1):
        wid = lax.axis_index("s") * info.num_cores + lax.axis_index("c")
        base = pl.multiple_of(wid * b_per_w, b_per_w)
        pltpu.sync_copy(idx_hbm.at[pl.ds(base, b_per_w)], idx_v)

        def body(i, carry):
            o = pl.multiple_of(i * 2 * ch, 2 * ch)
            cp0 = pltpu.async_copy(
                z_hbm.at[idx_v.at[pl.ds(o, ch)]], row0_v, sem0)
            cp1 = pltpu.async_copy(
                z_hbm.at[idx_v.at[pl.ds(o + ch, ch)]], row1_v, sem1)
            cp0.wait()
            pltpu.sync_copy(row0_v, out_hbm.at[pl.ds(base + o, ch)])
            cp1.wait()
            pltpu.sync_copy(row1_v, out_hbm.at[pl.ds(base + o + ch, ch)])
            return carry

        lax.fori_loop(0, n_pair, body, 0)

    return gather_kernel(z, idx)


def kernel(edge_fea, angle_fea, nbr_idx, W_lin, b_lin, W_att, b_att, g1, b1, g2, b2):
    W1, W2, W3 = W_lin[:F], W_lin[F:2 * F], W_lin[2 * F:]
    Wa1, Wa2, Wa3 = W_att[:F], W_att[F:2 * F], W_att[2 * F:]

    eye8 = jnp.eye(M, dtype=_f32)

    bd2 = jnp.kron(eye8, W2)

    c2 = jnp.linalg.solve(W2, Wa2)
    c2 = c2 + jnp.linalg.solve(W2, Wa2 - W2 @ c2)
    scm = jnp.kron(eye8, c2)

    bd3 = jnp.concatenate([jnp.kron(eye8, W3), jnp.kron(eye8, Wa3)], axis=1)

    w1t = jnp.concatenate([jnp.tile(W1, (1, M)), jnp.tile(Wa1, (1, M))], axis=1)

    b8g = jnp.concatenate([_B8 * jnp.tile(g1, M)[None, :], jnp.asarray(_B8)], axis=0)

    row3 = jnp.concatenate([g2, b2, b_att, jnp.zeros((MF - 2 * F - 1,), _f32)])
    cv = jnp.stack([jnp.tile(b_lin, M), jnp.tile(g1, M), jnp.tile(b1, M), row3])

    e_flat = edge_fea.reshape(NE, F)
    ang3 = angle_fea.reshape(NE, M, A)
    idx_flat = nbr_idx.reshape(NE)

    bn = 2048
    z = pl.pallas_call(
        _phase0_body,
        grid=(N // bn,),
        in_specs=[
            pl.BlockSpec((bn, M, F), lambda i: (i, 0, 0)),
            pl.BlockSpec((MF, MF), lambda i: (0, 0)),
        ],
        out_specs=pl.BlockSpec((bn, D), lambda i: (i, 0)),
        out_shape=jax.ShapeDtypeStruct((N, D), jnp.int32),
    )(edge_fea, bd2)

    gz = _sc_gather(z, idx_flat)

    r = 4096
    out = pl.pallas_call(
        _main_body,
        grid=(NE // r,),
        in_specs=[
            pl.BlockSpec((r, F), lambda i: (i, 0)),
            pl.BlockSpec((r, D), lambda i: (i, 0)),
            pl.BlockSpec((r, M, A), lambda i: (i, 0, 0)),
            pl.BlockSpec((F, MF + M), lambda i: (0, 0)),
            pl.BlockSpec((MA, MF + M), lambda i: (0, 0)),
            pl.BlockSpec((MF, M), lambda i: (0, 0)),
            pl.BlockSpec((2 * M, MF), lambda i: (0, 0)),
            pl.BlockSpec((MF, F), lambda i: (0, 0)),
            pl.BlockSpec((MF, M), lambda i: (0, 0)),
            pl.BlockSpec((4, MF), lambda i: (0, 0)),
        ],
        out_specs=pl.BlockSpec((r, F), lambda i: (i, 0)),
        out_shape=jax.ShapeDtypeStruct((NE, F), _f32),
    )(e_flat, gz, ang3, w1t, bd3, _S, b8g, _KS, scm, cv)

    return out.reshape(N, M, F)


# speedup vs baseline: 1.3863x; 1.1810x over previous
import functools

import numpy as np
import jax
import jax.numpy as jnp
from jax import lax
from jax.experimental import pallas as pl
from jax.experimental.pallas import tpu as pltpu
from jax.experimental.pallas import tpu_sc as plsc

N, M, F, A = 16384, 8, 32, 16
MF = M * F
MA = M * A
D = MF // 2
NE = N * M

_f32 = jnp.float32

_S = np.zeros((MF, M), np.float32)
_B8 = np.zeros((M, MF), np.float32)
_KS = np.zeros((MF, F), np.float32)
for _k in range(M):
    _S[_k * F:(_k + 1) * F, _k] = 1.0
    _B8[_k, _k * F:(_k + 1) * F] = 1.0
    _KS[_k * F:(_k + 1) * F, :] = np.eye(F, dtype=np.float32)


def _phase0_body(e_ref, w_ref, o_ref):
    e2 = e_ref[...].reshape(e_ref.shape[0], MF)
    y = jnp.dot(e2, w_ref[...], preferred_element_type=_f32)
    lo = lax.bitcast_convert_type(y[:, :MF // 2].astype(jnp.bfloat16),
                                  jnp.uint16).astype(jnp.uint32)
    hi = lax.bitcast_convert_type(y[:, MF // 2:].astype(jnp.bfloat16),
                                  jnp.uint16).astype(jnp.uint32)
    o_ref[...] = lax.bitcast_convert_type(lo | (hi << 16), jnp.int32)


def _main_body(e_ref, gz_ref, ang_ref, w1t_ref, bd3_ref, s_ref, b8g_ref,
               ks_ref, scm_ref, cv_ref, o_ref):
    e = e_ref[...]
    gzu = lax.bitcast_convert_type(gz_ref[...], jnp.uint32)
    gz = jnp.concatenate(
        [lax.bitcast_convert_type(gzu << 16, _f32),
         lax.bitcast_convert_type(gzu & jnp.uint32(0xFFFF0000), _f32)],
        axis=1)
    cv = cv_ref[...]
    blin_t = cv[0:1, :]
    b1_t = cv[2:3, :]
    g2 = cv[3:4, 0:F]
    b2 = cv[3:4, F:2 * F]
    b_att = cv[3:4, 2 * F:2 * F + 1]

    ang = ang_ref[...].reshape(ang_ref.shape[0] * M, MA)
    t = (jnp.dot(e, w1t_ref[...], preferred_element_type=_f32)
         + jnp.dot(ang, bd3_ref[...], preferred_element_type=_f32))
    lin = gz + t[:, :MF] + blin_t
    att = (t[:, MF:MF + M] + b_att
           + jnp.dot(gz, scm_ref[...], preferred_element_type=_f32))
    att = jnp.where(att >= 0, att, 0.01 * att)
    ex = jnp.exp(att)
    alpha = ex / jnp.sum(ex, axis=1, keepdims=True)

    b8g = b8g_ref[...]
    x = jnp.dot(alpha, b8g[M:M + M, :], preferred_element_type=_f32) * lin
    s = s_ref[...]
    mu = jnp.dot(x, s, preferred_element_type=_f32) * (1.0 / F)
    msq = jnp.dot(x * x, s, preferred_element_type=_f32) * (1.0 / F)
    inv = lax.rsqrt(msq - mu * mu + 1e-5)
    a_sc = jnp.dot(inv, b8g[:M, :], preferred_element_type=_f32)
    c_sc = b1_t - jnp.dot(mu * inv, b8g[:M, :], preferred_element_type=_f32)
    xn = x * a_sc + c_sc
    x = jnp.log1p(jnp.exp(xn))

    y = e + jnp.dot(x, ks_ref[...], preferred_element_type=_f32)
    mu2 = jnp.mean(y, axis=1, keepdims=True)
    yc = y - mu2
    var2 = jnp.mean(yc * yc, axis=1, keepdims=True)
    y = yc * lax.rsqrt(var2 + 1e-5) * g2 + b2
    o_ref[...] = jnp.log1p(jnp.exp(y))


def _sc_gather(z, idx):
    nidx = idx.shape[0]
    zdt = z.dtype
    info = plsc.get_sparse_core_info()
    nw = info.num_cores * info.num_subcores
    b_per_w = nidx // nw
    ch = 128
    n_pair = b_per_w // (2 * ch)
    mesh = plsc.VectorSubcoreMesh(core_axis_name="c", subcore_axis_name="s")

    @functools.partial(
        pl.kernel, mesh=mesh,
        out_type=jax.ShapeDtypeStruct((nidx, D), zdt),
        scratch_types=[
            pltpu.VMEM((b_per_w,), jnp.int32),
            pltpu.VMEM((ch, D), zdt),
            pltpu.VMEM((ch, D), zdt),
            pltpu.SemaphoreType.DMA,
            pltpu.SemaphoreType.DMA,
        ],
    )
    def gather_kernel(z_hbm, idx_hbm, out_hbm, idx_v, row0_v, row1_v, sem0, sem1):
        wid = lax.axis_index("s") * info.num_cores + lax.axis_index("c")
        base = pl.multiple_of(wid * b_per_w, b_per_w)
        pltpu.sync_copy(idx_hbm.at[pl.ds(base, b_per_w)], idx_v)

        def body(i, carry):
            o = pl.multiple_of(i * 2 * ch, 2 * ch)
            cp0 = pltpu.async_copy(
                z_hbm.at[idx_v.at[pl.ds(o, ch)]], row0_v, sem0)
            cp1 = pltpu.async_copy(
                z_hbm.at[idx_v.at[pl.ds(o + ch, ch)]], row1_v, sem1)
            cp0.wait()
            pltpu.sync_copy(row0_v, out_hbm.at[pl.ds(base + o, ch)])
            cp1.wait()
            pltpu.sync_copy(row1_v, out_hbm.at[pl.ds(base + o + ch, ch)])
            return carry

        lax.fori_loop(0, n_pair, body, 0)

    return gather_kernel(z, idx)


def kernel(edge_fea, angle_fea, nbr_idx, W_lin, b_lin, W_att, b_att, g1, b1, g2, b2):
    W1, W2, W3 = W_lin[:F], W_lin[F:2 * F], W_lin[2 * F:]
    Wa1, Wa2, Wa3 = W_att[:F], W_att[F:2 * F], W_att[2 * F:]

    eye8 = jnp.eye(M, dtype=_f32)

    bd2 = jnp.kron(eye8, W2)

    c2 = jnp.linalg.solve(W2, Wa2)
    c2 = c2 + jnp.linalg.solve(W2, Wa2 - W2 @ c2)
    scm = jnp.kron(eye8, c2)

    bd3 = jnp.concatenate([jnp.kron(eye8, W3), jnp.kron(eye8, Wa3)], axis=1)

    w1t = jnp.concatenate([jnp.tile(W1, (1, M)), jnp.tile(Wa1, (1, M))], axis=1)

    b8g = jnp.concatenate([_B8 * jnp.tile(g1, M)[None, :], jnp.asarray(_B8)], axis=0)

    row3 = jnp.concatenate([g2, b2, b_att, jnp.zeros((MF - 2 * F - 1,), _f32)])
    cv = jnp.stack([jnp.tile(b_lin, M), jnp.tile(g1, M), jnp.tile(b1, M), row3])

    e_flat = edge_fea.reshape(NE, F)
    ang128 = angle_fea.reshape(N, M, MA)
    idx_flat = nbr_idx.reshape(NE)

    bn = 2048
    z = pl.pallas_call(
        _phase0_body,
        grid=(N // bn,),
        in_specs=[
            pl.BlockSpec((bn, M, F), lambda i: (i, 0, 0)),
            pl.BlockSpec((MF, MF), lambda i: (0, 0)),
        ],
        out_specs=pl.BlockSpec((bn, D), lambda i: (i, 0)),
        out_shape=jax.ShapeDtypeStruct((N, D), jnp.int32),
    )(edge_fea, bd2)

    gz = _sc_gather(z, idx_flat)

    r = 4096
    out = pl.pallas_call(
        _main_body,
        grid=(NE // r,),
        in_specs=[
            pl.BlockSpec((r, F), lambda i: (i, 0)),
            pl.BlockSpec((r, D), lambda i: (i, 0)),
            pl.BlockSpec((r // M, M, MA), lambda i: (i, 0, 0)),
            pl.BlockSpec((F, MF + M), lambda i: (0, 0)),
            pl.BlockSpec((MA, MF + M), lambda i: (0, 0)),
            pl.BlockSpec((MF, M), lambda i: (0, 0)),
            pl.BlockSpec((2 * M, MF), lambda i: (0, 0)),
            pl.BlockSpec((MF, F), lambda i: (0, 0)),
            pl.BlockSpec((MF, M), lambda i: (0, 0)),
            pl.BlockSpec((4, MF), lambda i: (0, 0)),
        ],
        out_specs=pl.BlockSpec((r, F), lambda i: (i, 0)),
        out_shape=jax.ShapeDtypeStruct((NE, F), _f32),
    )(e_flat, gz, ang128, w1t, bd3, _S, b8g, _KS, scm, cv)

    return out.reshape(N, M, F)
